# Initial kernel scaffold; baseline (speedup 1.0000x reference)
import functools

import jax
import jax.numpy as jnp
from jax import lax
from jax.experimental import pallas as pl
from jax.experimental.pallas import tpu as pltpu
from jax.experimental.pallas import tpu_sc as plsc

VOCAB = 1000
D = 1000
NC = 2
NS = 16
NW = NC * NS

CHUNK = 64


def _gather_body(idx_hbm, table_hbm, out_hbm, idx_v, rows_v, gsem):
    b_per_w = idx_hbm.shape[0] // NW
    n_chunks = b_per_w // CHUNK
    wid = lax.axis_index("s") * NC + lax.axis_index("c")
    base = wid * b_per_w
    pltpu.sync_copy(idx_hbm.at[pl.ds(base, b_per_w)], idx_v)

    def chunk_step(c, carry):
        r0 = pl.multiple_of(c * CHUNK, CHUNK)
        pltpu.async_copy(
            table_hbm.at[idx_v.at[pl.ds(r0, CHUNK)]], rows_v, gsem
        ).wait()
        pltpu.sync_copy(rows_v, out_hbm.at[pl.ds(base + r0, CHUNK)])
        return carry

    lax.fori_loop(0, n_chunks, chunk_step, 0)


def _gather(idx, table):
    b = idx.shape[0]
    mesh = plsc.VectorSubcoreMesh(
        core_axis_name="c", subcore_axis_name="s", num_cores=NC,
        num_subcores=NS,
    )
    run = pl.kernel(
        _gather_body,
        out_type=jax.ShapeDtypeStruct((b, D), jnp.float32),
        mesh=mesh,
        scratch_types=[
            pltpu.VMEM((b // NW,), jnp.int32),
            pltpu.VMEM((CHUNK, D), jnp.float32),
            pltpu.SemaphoreType.DMA,
        ],
    )
    return run(idx, table)


def kernel(x, table):
    bsz, seq = x.shape
    idx = x.reshape(bsz * seq)
    out = _gather(idx, table)
    return out.reshape(bsz, seq, VOCAB)


# speedup vs baseline: 1.3903x; 1.3903x over previous
import functools

import jax
import jax.numpy as jnp
from jax import lax
from jax.experimental import pallas as pl
from jax.experimental.pallas import tpu as pltpu
from jax.experimental.pallas import tpu_sc as plsc

VOCAB = 1000
D = 1000
NC = 2
NS = 16
NW = NC * NS

CHUNK = 64


DPAD = 1024


def _gather_body(idx_hbm, table_hbm, out_hbm, idx_v, rows_v, gsem):
    b_per_w = idx_hbm.shape[0] // NW
    n_chunks = b_per_w // CHUNK
    wid = lax.axis_index("s") * NC + lax.axis_index("c")
    base = wid * b_per_w
    pltpu.sync_copy(idx_hbm.at[pl.ds(base, b_per_w)], idx_v)

    def chunk_step(c, carry):
        r0 = pl.multiple_of(c * CHUNK, CHUNK)
        pltpu.async_copy(
            table_hbm.at[idx_v.at[pl.ds(r0, CHUNK)]], rows_v, gsem
        ).wait()
        pltpu.sync_copy(rows_v, out_hbm.at[pl.ds(base + r0, CHUNK)])
        return carry

    lax.fori_loop(0, n_chunks, chunk_step, 0)


def _gather(idx, table):
    b = idx.shape[0]
    mesh = plsc.VectorSubcoreMesh(
        core_axis_name="c", subcore_axis_name="s", num_cores=NC,
        num_subcores=NS,
    )
    run = pl.kernel(
        _gather_body,
        out_type=jax.ShapeDtypeStruct((b, DPAD), jnp.float32),
        mesh=mesh,
        scratch_types=[
            pltpu.VMEM((b // NW,), jnp.int32),
            pltpu.VMEM((CHUNK, DPAD), jnp.float32),
            pltpu.SemaphoreType.DMA,
        ],
    )
    return run(idx, table)


def kernel(x, table):
    bsz, seq = x.shape
    idx = x.reshape(bsz * seq)
    table_p = jnp.pad(table, ((0, 0), (0, DPAD - D)))
    out = _gather(idx, table_p)
    return out[:, :D].reshape(bsz, seq, VOCAB)
